# baseline (speedup 1.0000x reference)
import functools

import jax
import jax.numpy as jnp
import numpy as np
from jax import lax
from jax.experimental import pallas as pl
from jax.experimental.pallas import tpu as pltpu
from jax.experimental.pallas import tpu_sc as plsc

N_C = 64
N_I = 128
N_OUT = N_C + N_I
NB = N_C - 1

NUM_CORES = 2
NUM_SUBCORES = 16
NW = NUM_CORES * NUM_SUBCORES
CHUNK = 128

_INV127 = np.float32(1.0) / np.float32(127.0)


def _sc_body(zv_hbm, wt_hbm, out_hbm, zv_v, wt_v, out_v, cdf_v, zmid_v,
             hist_v):
    n_rays = zv_hbm.shape[0]
    rays_per_w = n_rays // NW
    n_chunks = rays_per_w // CHUNK
    cid = lax.axis_index("c")
    sid = lax.axis_index("s")
    wid = sid * NUM_CORES + cid
    iota = lax.iota(jnp.int32, 16)
    f32 = jnp.float32
    i32 = jnp.int32

    def per_ray(r):
        rr = jnp.full((16,), r, i32)
        ones = jnp.ones((16,), i32)
        zA = [zv_v[r, pl.ds(16 * i, 16)] for i in range(4)]
        zB = [plsc.load_gather(
                  zv_v, [rr, jnp.minimum(iota + (16 * i + 1), N_C - 1)])
              for i in range(4)]
        wB = [plsc.load_gather(
                  wt_v, [rr, jnp.minimum(iota + (16 * i + 1), N_C - 1)])
              for i in range(4)]
        for i in range(4):
            zmid_v[r, pl.ds(16 * i, 16)] = f32(0.5) * (zA[i] + zB[i])
        carry = f32(0.0)
        csum = []
        for i in range(4):
            w = wB[i] + f32(1e-5)
            if i == 3:
                w = jnp.where(iota < 14, w, f32(0.0))
            c = plsc.cumsum(w) + carry
            csum.append(c)
            carry = jnp.max(c)
        rS = f32(1.0) / jnp.full((16,), carry, f32)
        cdf_v[r, pl.ds(0, 16)] = jnp.zeros((16,), f32)
        cdfn = []
        for i in range(4):
            cn = csum[i] * rS
            cdfn.append(cn)
            m = (iota < 14) if i == 3 else None
            plsc.store_scatter(cdf_v, [rr, iota + (16 * i + 1)], cn, mask=m)
        for i in range(8):
            hist_v[r, pl.ds(16 * i, 16)] = jnp.zeros((16,), i32)
        for i in range(4):
            x = f32(127.0) * cdfn[i]
            ti = x.astype(i32)
            s = ti + (ti.astype(f32) < x).astype(i32)
            m = s <= 127
            if i == 3:
                m = m & (iota < 14)
            plsc.addupdate_scatter(hist_v, [rr, jnp.minimum(s, 127)], ones,
                                   mask=m)
        icarry = i32(0)
        fcarry = f32(-1e30)
        zs = []
        bels = []
        for i in range(8):
            h = hist_v[r, pl.ds(16 * i, 16)]
            below = plsc.cumsum(h) + icarry
            bels.append(below)
            icarry = jnp.max(below)
            above = jnp.minimum(below + 1, NB - 1)
            c0 = plsc.load_gather(cdf_v, [rr, below])
            c1 = plsc.load_gather(cdf_v, [rr, above])
            b0 = plsc.load_gather(zmid_v, [rr, below])
            b1 = plsc.load_gather(zmid_v, [rr, above])
            u = (iota + (16 * i)).astype(f32) * _INV127
            den = c1 - c0
            den = jnp.where(den < f32(1e-5), f32(1.0), den)
            t = (u - c0) / den
            z = b0 + t * (b1 - b0)
            z = plsc.cummax(z)
            z = jnp.maximum(z, fcarry)
            fcarry = jnp.max(z)
            zs.append(z)
        for i in range(5):
            hist_v[r, pl.ds(16 * i, 16)] = jnp.zeros((16,), i32)
        for i in range(8):
            b = zs[i]
            znx = plsc.load_gather(zv_v, [rr, bels[i] + 1])
            cntA = bels[i] + 1 + (znx <= b).astype(i32)
            plsc.addupdate_scatter(hist_v, [rr, cntA], ones)
            plsc.store_scatter(out_v, [rr, iota + (16 * i) + cntA], b)
        icarry = i32(0)
        for i in range(4):
            h = hist_v[r, pl.ds(16 * i, 16)]
            pc = plsc.cumsum(h) + icarry
            icarry = jnp.max(pc)
            plsc.store_scatter(out_v, [rr, iota + (16 * i) + pc], zA[i])

    def per_chunk(ch, _):
        base = wid * rays_per_w + ch * CHUNK
        pltpu.sync_copy(zv_hbm.at[pl.ds(base, CHUNK)], zv_v)
        pltpu.sync_copy(wt_hbm.at[pl.ds(base, CHUNK)], wt_v)

        def ray_loop(r, _):
            per_ray(r)
            return 0

        lax.fori_loop(0, CHUNK, ray_loop, 0)

        pltpu.sync_copy(out_v, out_hbm.at[pl.ds(base, CHUNK)])
        return 0

    lax.fori_loop(0, n_chunks, per_chunk, 0)


def _make_sc_kernel(n_rays):
    mesh = plsc.VectorSubcoreMesh(core_axis_name="c", subcore_axis_name="s",
                                  num_cores=NUM_CORES,
                                  num_subcores=NUM_SUBCORES)
    return pl.kernel(
        _sc_body,
        out_type=jax.ShapeDtypeStruct((n_rays, N_OUT), jnp.float32),
        mesh=mesh,
        compiler_params=pltpu.CompilerParams(needs_layout_passes=False),
        scratch_types=[
            pltpu.VMEM((CHUNK, N_C), jnp.float32),
            pltpu.VMEM((CHUNK, N_C), jnp.float32),
            pltpu.VMEM((CHUNK, N_OUT), jnp.float32),
            pltpu.VMEM((CHUNK, N_C), jnp.float32),
            pltpu.VMEM((CHUNK, N_C), jnp.float32),
            pltpu.VMEM((CHUNK, 128), jnp.int32),
        ],
    )


_PTS_R = 512


def _pts_body(o_ref, d_ref, z_ref, out_ref, zout_ref):
    i32 = jnp.int32
    f32 = jnp.float32
    lane = lax.broadcasted_iota(i32, (N_OUT, 3 * N_OUT), 1)
    row = lax.broadcasted_iota(i32, (N_OUT, 3 * N_OUT), 0)
    zsel = (lane // 3 == row).astype(f32)
    lane3 = lax.broadcasted_iota(i32, (8, 3 * N_OUT), 1)
    row3 = lax.broadcasted_iota(i32, (8, 3 * N_OUT), 0)
    csel = (lane3 % 3 == row3 % 3).astype(f32) * (row3 < 3).astype(f32)
    og = jnp.dot(o_ref[...], csel, preferred_element_type=f32)
    dg = jnp.dot(d_ref[...], csel, preferred_element_type=f32)
    zg = jnp.dot(z_ref[...], zsel, preferred_element_type=f32)
    out_ref[...] = og + dg * zg
    zout_ref[...] = z_ref[...]


def _pts_call(rays_o, rays_d, z_all):
    n = z_all.shape[0]
    grid = n // _PTS_R
    return pl.pallas_call(
        _pts_body,
        grid=(grid,),
        in_specs=[
            pl.BlockSpec((_PTS_R, 8), lambda i: (i, 0)),
            pl.BlockSpec((_PTS_R, 8), lambda i: (i, 0)),
            pl.BlockSpec((_PTS_R, N_OUT), lambda i: (i, 0)),
        ],
        out_specs=[pl.BlockSpec((_PTS_R, 3 * N_OUT), lambda i: (i, 0)),
                   pl.BlockSpec((_PTS_R, N_OUT), lambda i: (i, 0))],
        out_shape=[jax.ShapeDtypeStruct((n, 3 * N_OUT), jnp.float32),
                   jax.ShapeDtypeStruct((n, N_OUT), jnp.float32)],
        compiler_params=pltpu.CompilerParams(
            dimension_semantics=("arbitrary",)),
    )(jnp.pad(rays_o, ((0, 0), (0, 5))), jnp.pad(rays_d, ((0, 0), (0, 5))),
      z_all)


def kernel(rays_o, rays_d, z_vals, weights):
    n = z_vals.shape[0]
    z_all = _make_sc_kernel(n)(z_vals, weights)
    pts_flat, z_out = _pts_call(rays_o, rays_d, z_all)
    return pts_flat.reshape(n, N_OUT, 3), z_out


# speedup vs baseline: 1.0322x; 1.0322x over previous
import functools

import jax
import jax.numpy as jnp
import numpy as np
from jax import lax
from jax.experimental import pallas as pl
from jax.experimental.pallas import tpu as pltpu
from jax.experimental.pallas import tpu_sc as plsc

N_C = 64
N_I = 128
N_OUT = N_C + N_I
NB = N_C - 1

NUM_CORES = 2
NUM_SUBCORES = 16
NW = NUM_CORES * NUM_SUBCORES
CHUNK = 128

_INV127 = np.float32(1.0) / np.float32(127.0)


def _sc_body(zv_hbm, wt_hbm, out_hbm, zv_v, wt_v, out_v, cdf_v0, zmid_v0,
             hist_v0, cdf_v1, zmid_v1, hist_v1):
    n_rays = zv_hbm.shape[0]
    rays_per_w = n_rays // NW
    n_chunks = rays_per_w // CHUNK
    cid = lax.axis_index("c")
    sid = lax.axis_index("s")
    wid = sid * NUM_CORES + cid
    iota = lax.iota(jnp.int32, 16)
    f32 = jnp.float32
    i32 = jnp.int32

    def per_ray(r, cdf_v, zmid_v, hist_v):
        rr = jnp.full((16,), r, i32)
        zA = [zv_v[r, pl.ds(16 * i, 16)] for i in range(4)]
        zB = [plsc.load_gather(
                  zv_v, [rr, jnp.minimum(iota + (16 * i + 1), N_C - 1)])
              for i in range(4)]
        wB = [plsc.load_gather(
                  wt_v, [rr, jnp.minimum(iota + (16 * i + 1), N_C - 1)])
              for i in range(4)]
        for i in range(4):
            zmid_v[pl.ds(16 * i, 16)] = f32(0.5) * (zA[i] + zB[i])
        carry = f32(0.0)
        csum = []
        for i in range(4):
            w = wB[i] + f32(1e-5)
            if i == 3:
                w = jnp.where(iota < 14, w, f32(0.0))
            c = plsc.cumsum(w) + carry
            csum.append(c)
            carry = jnp.max(c)
        rS = f32(1.0) / jnp.full((16,), carry, f32)
        cdf_v[pl.ds(0, 16)] = jnp.zeros((16,), f32)
        cdfn = []
        for i in range(4):
            cn = csum[i] * rS
            cdfn.append(cn)
            m = (iota < 14) if i == 3 else None
            plsc.store_scatter(cdf_v, [iota + (16 * i + 1)], cn, mask=m)
        for i in range(8):
            hist_v[pl.ds(16 * i, 16)] = jnp.zeros((16,), i32)
        ones = jnp.ones((16,), i32)
        for i in range(4):
            x = f32(127.0) * cdfn[i]
            ti = x.astype(i32)
            s = ti + (ti.astype(f32) < x).astype(i32)
            m = s <= 127
            if i == 3:
                m = m & (iota < 14)
            plsc.addupdate_scatter(hist_v, [jnp.minimum(s, 127)], ones, mask=m)
        icarry = i32(0)
        fcarry = f32(-1e30)
        zs = []
        bels = []
        for i in range(8):
            h = hist_v[pl.ds(16 * i, 16)]
            below = plsc.cumsum(h) + icarry
            bels.append(below)
            icarry = jnp.max(below)
            above = jnp.minimum(below + 1, NB - 1)
            c0 = plsc.load_gather(cdf_v, [below])
            c1 = plsc.load_gather(cdf_v, [above])
            b0 = plsc.load_gather(zmid_v, [below])
            b1 = plsc.load_gather(zmid_v, [above])
            u = (iota + (16 * i)).astype(f32) * _INV127
            den = c1 - c0
            den = jnp.where(den < f32(1e-5), f32(1.0), den)
            t = (u - c0) / den
            z = b0 + t * (b1 - b0)
            z = plsc.cummax(z)
            z = jnp.maximum(z, fcarry)
            fcarry = jnp.max(z)
            zs.append(z)
        for i in range(5):
            hist_v[pl.ds(16 * i, 16)] = jnp.zeros((16,), i32)
        for i in range(8):
            b = zs[i]
            znx = plsc.load_gather(zv_v, [rr, bels[i] + 1])
            cntA = bels[i] + 1 + (znx <= b).astype(i32)
            plsc.addupdate_scatter(hist_v, [cntA], ones)
            plsc.store_scatter(out_v, [rr, iota + (16 * i) + cntA], b)
        icarry = i32(0)
        for i in range(4):
            h = hist_v[pl.ds(16 * i, 16)]
            pc = plsc.cumsum(h) + icarry
            icarry = jnp.max(pc)
            plsc.store_scatter(out_v, [rr, iota + (16 * i) + pc], zA[i])

    def per_pair(q, _):
        per_ray(2 * q, cdf_v0, zmid_v0, hist_v0)
        per_ray(2 * q + 1, cdf_v1, zmid_v1, hist_v1)
        return 0

    def per_chunk(ch, _):
        base = wid * rays_per_w + ch * CHUNK
        pltpu.sync_copy(zv_hbm.at[pl.ds(base, CHUNK)], zv_v)
        pltpu.sync_copy(wt_hbm.at[pl.ds(base, CHUNK)], wt_v)
        lax.fori_loop(0, CHUNK // 2, per_pair, 0)
        pltpu.sync_copy(out_v, out_hbm.at[pl.ds(base, CHUNK)])
        return 0

    lax.fori_loop(0, n_chunks, per_chunk, 0)


def _make_sc_kernel(n_rays):
    mesh = plsc.VectorSubcoreMesh(core_axis_name="c", subcore_axis_name="s",
                                  num_cores=NUM_CORES,
                                  num_subcores=NUM_SUBCORES)
    return pl.kernel(
        _sc_body,
        out_type=jax.ShapeDtypeStruct((n_rays, N_OUT), jnp.float32),
        mesh=mesh,
        compiler_params=pltpu.CompilerParams(needs_layout_passes=False),
        scratch_types=[
            pltpu.VMEM((CHUNK, N_C), jnp.float32),
            pltpu.VMEM((CHUNK, N_C), jnp.float32),
            pltpu.VMEM((CHUNK, N_OUT), jnp.float32),
            pltpu.VMEM((64,), jnp.float32),
            pltpu.VMEM((64,), jnp.float32),
            pltpu.VMEM((128,), jnp.int32),
            pltpu.VMEM((64,), jnp.float32),
            pltpu.VMEM((64,), jnp.float32),
            pltpu.VMEM((128,), jnp.int32),
        ],
    )


_PTS_R = 512


def _pts_body(o_ref, d_ref, z_ref, out_ref):
    i32 = jnp.int32
    f32 = jnp.float32
    lane = lax.broadcasted_iota(i32, (N_OUT, 3 * N_OUT), 1)
    row = lax.broadcasted_iota(i32, (N_OUT, 3 * N_OUT), 0)
    zsel = (lane // 3 == row).astype(f32)
    lane3 = lax.broadcasted_iota(i32, (8, 3 * N_OUT), 1)
    row3 = lax.broadcasted_iota(i32, (8, 3 * N_OUT), 0)
    csel = (lane3 % 3 == row3 % 3).astype(f32) * (row3 < 3).astype(f32)
    og = jnp.dot(o_ref[...], csel, preferred_element_type=f32)
    dg = jnp.dot(d_ref[...], csel, preferred_element_type=f32)
    zg = jnp.dot(z_ref[...], zsel, preferred_element_type=f32)
    out_ref[...] = og + dg * zg


def _pts_call(rays_o, rays_d, z_all):
    n = z_all.shape[0]
    grid = n // _PTS_R
    return pl.pallas_call(
        _pts_body,
        grid=(grid,),
        in_specs=[
            pl.BlockSpec((_PTS_R, 8), lambda i: (i, 0)),
            pl.BlockSpec((_PTS_R, 8), lambda i: (i, 0)),
            pl.BlockSpec((_PTS_R, N_OUT), lambda i: (i, 0)),
        ],
        out_specs=pl.BlockSpec((_PTS_R, 3 * N_OUT), lambda i: (i, 0)),
        out_shape=jax.ShapeDtypeStruct((n, 3 * N_OUT), jnp.float32),
        compiler_params=pltpu.CompilerParams(
            dimension_semantics=("arbitrary",)),
    )(jnp.pad(rays_o, ((0, 0), (0, 5))), jnp.pad(rays_d, ((0, 0), (0, 5))),
      z_all)


def kernel(rays_o, rays_d, z_vals, weights):
    n = z_vals.shape[0]
    z_all = _make_sc_kernel(n)(z_vals, weights)
    pts = _pts_call(rays_o, rays_d, z_all).reshape(n, N_OUT, 3)
    return pts, z_all
